# baseline (speedup 1.0000x reference)
import functools

import jax
import jax.numpy as jnp
from jax import lax
from jax.experimental import pallas as pl
from jax.experimental.pallas import tpu as pltpu
from jax.experimental.pallas import tpu_sc as plsc

S = 8192
D = 2048
NC = 2
NS = 16
L = 16
NW = NC * NS
CHUNK = S // NW
NV = CHUNK // L
B = 8
NLR = CHUNK // B
NBUF = 5


def _body(inputs_hbm, mask_hbm, source_hbm, out_hbm,
          mask_v, buf0, buf1, buf2, buf3, buf4,
          mlist_src, mlist_pos, ulist_gidx, ulist_pos,
          sem_g0, sem_g1, sem_g2, sem_g3, sem_g4,
          sem_s0, sem_s1, sem_s2, sem_s3, sem_s4):
    wid = lax.axis_index("s") * NC + lax.axis_index("c")
    base = wid * CHUNK

    pltpu.sync_copy(mask_hbm, mask_v)

    iota = lax.iota(jnp.int32, L)
    zero16 = jnp.zeros((L,), jnp.int32)

    off_m = zero16
    off_u = zero16
    for j in range(NV):
        v = mask_v[pl.ds(base + j * L, L)]
        m = v > 0
        um = jnp.logical_not(m)
        cs = plsc.cumsum(v)
        pcnt = plsc.all_reduce_population_count(m)
        p = base + j * L + iota
        dest = off_m + cs - 1
        plsc.store_scatter(mlist_src, [dest >> 3, dest & 7], dest, mask=m)
        plsc.store_scatter(mlist_pos, [dest >> 3, dest & 7], p, mask=m)
        dest_u = off_u + plsc.cumsum(1 - v) - 1
        plsc.store_scatter(ulist_gidx, [dest_u >> 3, dest_u & 7], p, mask=um)
        plsc.store_scatter(ulist_pos, [dest_u >> 3, dest_u & 7], p, mask=um)
        off_m = off_m + pcnt
        off_u = off_u + (L - pcnt)

    nm = jnp.max(off_m, axis=0)
    nu = CHUNK - nm

    def tail_fix(lst_a, lst_b, n):
        @pl.when(lax.rem(n, B) != 0)
        def _():
            full = n >> 3
            tsel = lax.rem(full * B + iota, jnp.full((L,), n, jnp.int32))
            row = tsel >> 3
            col = tsel & 7
            lane_ok = iota < B
            full_b = jnp.full((L,), full, jnp.int32)
            plsc.store_scatter(lst_a, [full_b, iota],
                               plsc.load_gather(lst_a, [row, col]),
                               mask=lane_ok)
            plsc.store_scatter(lst_b, [full_b, iota],
                               plsc.load_gather(lst_b, [row, col]),
                               mask=lane_ok)

    tail_fix(ulist_gidx, ulist_pos, nu)
    tail_fix(mlist_src, mlist_pos, nm)

    n_mb = (nm + (B - 1)) >> 3
    n_ub = (nu + (B - 1)) >> 3
    nt = n_ub + n_mb

    bufs = (buf0, buf1, buf2, buf3, buf4)
    gsems = (sem_g0, sem_g1, sem_g2, sem_g3, sem_g4)
    ssems = (sem_s0, sem_s1, sem_s2, sem_s3, sem_s4)

    def gat(g, k):
        @pl.when(g < n_ub)
        def _():
            pltpu.async_copy(inputs_hbm.at[ulist_gidx.at[g]], bufs[k],
                             gsems[k])

        @pl.when(g >= n_ub)
        def _():
            pltpu.async_copy(source_hbm.at[mlist_src.at[g - n_ub]], bufs[k],
                             gsems[k])

    def wgat(k):
        pltpu.make_async_copy(inputs_hbm.at[ulist_gidx.at[0]], bufs[k],
                              gsems[k]).wait()

    def sct(g, k):
        @pl.when(g < n_ub)
        def _():
            pltpu.async_copy(bufs[k], out_hbm.at[ulist_pos.at[g]], ssems[k])

        @pl.when(g >= n_ub)
        def _():
            pltpu.async_copy(bufs[k], out_hbm.at[mlist_pos.at[g - n_ub]],
                             ssems[k])

    def wsct(k):
        pltpu.make_async_copy(bufs[k], out_hbm.at[ulist_pos.at[0]],
                              ssems[k]).wait()

    for k in range(NBUF - 1):
        @pl.when(k < n_ub)
        def _():
            gat(k, k)

    def pf_body(j, acc):
        a = acc + mask_v[pl.ds(j * 4 * L, L)]
        a = a + mask_v[pl.ds((j * 4 + 1) * L, L)]
        a = a + mask_v[pl.ds((j * 4 + 2) * L, L)]
        return a + mask_v[pl.ds((j * 4 + 3) * L, L)]

    acc = lax.fori_loop(0, wid * (NV // 4), pf_body, zero16)
    prefix = jnp.full((L,), jnp.sum(acc, axis=0), jnp.int32)
    nmp = n_mb * B
    nmp_vec = jnp.full((L,), nmp, jnp.int32)
    for j2 in range(NV):
        @pl.when(j2 * L < nmp)
        def _():
            e = j2 * L + iota
            row = e >> 3
            col = e & 7
            plsc.store_scatter(
                mlist_src, [row, col],
                plsc.load_gather(mlist_src, [row, col]) + prefix,
                mask=e < nmp_vec)

    for k in range(NBUF - 1):
        @pl.when(k >= n_ub)
        def _():
            gat(k, k)

    def loop_body(t, _):
        for k in range(NBUF):
            g = NBUF * t + k

            @pl.when(g < nt)
            def _():
                @pl.when(g >= 1)
                def _():
                    wsct((k + NBUF - 1) % NBUF)

                @pl.when(g + NBUF - 1 < nt)
                def _():
                    gat(g + NBUF - 1, (k + NBUF - 1) % NBUF)
                wgat(k)
                sct(g, k)
        return 0

    lax.fori_loop(0, (nt + NBUF - 1) // NBUF, loop_body, 0)
    for k in range(NBUF):
        @pl.when(lax.rem(nt - 1, NBUF) == k)
        def _():
            wsct(k)


@functools.partial(
    pl.kernel,
    out_type=jax.ShapeDtypeStruct((S, D), jnp.float32),
    mesh=plsc.VectorSubcoreMesh(core_axis_name="c", subcore_axis_name="s"),
    compiler_params=pltpu.CompilerParams(needs_layout_passes=False),
    scratch_types=[
        pltpu.VMEM((S,), jnp.int32),
        pltpu.VMEM((B, D), jnp.float32),
        pltpu.VMEM((B, D), jnp.float32),
        pltpu.VMEM((B, D), jnp.float32),
        pltpu.VMEM((B, D), jnp.float32),
        pltpu.VMEM((B, D), jnp.float32),
        pltpu.VMEM((NLR, B), jnp.int32),
        pltpu.VMEM((NLR, B), jnp.int32),
        pltpu.VMEM((NLR, B), jnp.int32),
        pltpu.VMEM((NLR, B), jnp.int32),
        pltpu.SemaphoreType.DMA,
        pltpu.SemaphoreType.DMA,
        pltpu.SemaphoreType.DMA,
        pltpu.SemaphoreType.DMA,
        pltpu.SemaphoreType.DMA,
        pltpu.SemaphoreType.DMA,
        pltpu.SemaphoreType.DMA,
        pltpu.SemaphoreType.DMA,
        pltpu.SemaphoreType.DMA,
        pltpu.SemaphoreType.DMA,
    ],
)
def _sc_masked_scatter(inputs_hbm, mask_hbm, source_hbm, out_hbm, *scratch):
    _body(inputs_hbm, mask_hbm, source_hbm, out_hbm, *scratch)


def kernel(inputs_embeds, mask_1d, source):
    mask_i32 = mask_1d.astype(jnp.int32)
    return _sc_masked_scatter(inputs_embeds, mask_i32, source)


# speedup vs baseline: 1.0038x; 1.0038x over previous
import functools

import jax
import jax.numpy as jnp
from jax import lax
from jax.experimental import pallas as pl
from jax.experimental.pallas import tpu as pltpu
from jax.experimental.pallas import tpu_sc as plsc

S = 8192
D = 2048
NC = 2
NS = 16
L = 16
NW = NC * NS
CHUNK = S // NW
NV = CHUNK // L
B = 8
NLR = CHUNK // B
NBUF = 5


def _body(inputs_hbm, mask_hbm, source_hbm, out_hbm,
          mask_v, buf0, buf1, buf2, buf3, buf4,
          mlist_src, mlist_pos, ulist_gidx, ulist_pos,
          sem_g0, sem_g1, sem_g2, sem_g3, sem_g4,
          sem_s0, sem_s1, sem_s2, sem_s3, sem_s4):
    wid = lax.axis_index("s") * NC + lax.axis_index("c")
    base = wid * CHUNK

    pltpu.sync_copy(mask_hbm, mask_v)

    iota = lax.iota(jnp.int32, L)
    zero16 = jnp.zeros((L,), jnp.int32)

    def pf_body(j, acc):
        a = acc + mask_v[pl.ds(j * 4 * L, L)]
        a = a + mask_v[pl.ds((j * 4 + 1) * L, L)]
        a = a + mask_v[pl.ds((j * 4 + 2) * L, L)]
        return a + mask_v[pl.ds((j * 4 + 3) * L, L)]

    acc = lax.fori_loop(0, wid * (NV // 4), pf_body, zero16)
    carry = jnp.full((L,), jnp.sum(acc, axis=0), jnp.int32)

    off_m = zero16
    off_u = zero16
    for j in range(NV):
        v = mask_v[pl.ds(base + j * L, L)]
        m = v > 0
        um = jnp.logical_not(m)
        cs = plsc.cumsum(v)
        pcnt = plsc.all_reduce_population_count(m)
        src_idx = carry + cs - 1
        p = base + j * L + iota
        dest = off_m + cs - 1
        plsc.store_scatter(mlist_src, [dest >> 3, dest & 7], src_idx, mask=m)
        plsc.store_scatter(mlist_pos, [dest >> 3, dest & 7], p, mask=m)
        dest_u = off_u + plsc.cumsum(1 - v) - 1
        plsc.store_scatter(ulist_gidx, [dest_u >> 3, dest_u & 7], p, mask=um)
        plsc.store_scatter(ulist_pos, [dest_u >> 3, dest_u & 7], p, mask=um)
        off_m = off_m + pcnt
        off_u = off_u + (L - pcnt)
        carry = carry + pcnt

    nm = jnp.max(off_m, axis=0)
    nu = CHUNK - nm

    def tail_fix(lst_a, lst_b, n):
        @pl.when(lax.rem(n, B) != 0)
        def _():
            full = n >> 3
            tsel = lax.rem(full * B + iota, jnp.full((L,), n, jnp.int32))
            row = tsel >> 3
            col = tsel & 7
            lane_ok = iota < B
            full_b = jnp.full((L,), full, jnp.int32)
            plsc.store_scatter(lst_a, [full_b, iota],
                               plsc.load_gather(lst_a, [row, col]),
                               mask=lane_ok)
            plsc.store_scatter(lst_b, [full_b, iota],
                               plsc.load_gather(lst_b, [row, col]),
                               mask=lane_ok)

    tail_fix(ulist_gidx, ulist_pos, nu)
    tail_fix(mlist_src, mlist_pos, nm)

    n_mb = (nm + (B - 1)) >> 3
    n_ub = (nu + (B - 1)) >> 3
    nt = n_ub + n_mb

    bufs = (buf0, buf1, buf2, buf3, buf4)
    gsems = (sem_g0, sem_g1, sem_g2, sem_g3, sem_g4)
    ssems = (sem_s0, sem_s1, sem_s2, sem_s3, sem_s4)

    def gat(g, k):
        @pl.when(g < n_ub)
        def _():
            pltpu.async_copy(inputs_hbm.at[ulist_gidx.at[g]], bufs[k],
                             gsems[k])

        @pl.when(g >= n_ub)
        def _():
            pltpu.async_copy(source_hbm.at[mlist_src.at[g - n_ub]], bufs[k],
                             gsems[k])

    def wgat(k):
        pltpu.make_async_copy(inputs_hbm.at[ulist_gidx.at[0]], bufs[k],
                              gsems[k]).wait()

    def sct(g, k):
        @pl.when(g < n_ub)
        def _():
            pltpu.async_copy(bufs[k], out_hbm.at[ulist_pos.at[g]], ssems[k])

        @pl.when(g >= n_ub)
        def _():
            pltpu.async_copy(bufs[k], out_hbm.at[mlist_pos.at[g - n_ub]],
                             ssems[k])

    def wsct(k):
        pltpu.make_async_copy(bufs[k], out_hbm.at[ulist_pos.at[0]],
                              ssems[k]).wait()

    for k in range(NBUF - 1):
        gat(k, k)

    def loop_body(t, _):
        for k in range(NBUF):
            g = NBUF * t + k

            @pl.when(g < nt)
            def _():
                @pl.when(g >= 1)
                def _():
                    wsct((k + NBUF - 1) % NBUF)

                @pl.when(g + NBUF - 1 < nt)
                def _():
                    gat(g + NBUF - 1, (k + NBUF - 1) % NBUF)
                wgat(k)
                sct(g, k)
        return 0

    lax.fori_loop(0, (nt + NBUF - 1) // NBUF, loop_body, 0)
    for k in range(NBUF):
        @pl.when(lax.rem(nt - 1, NBUF) == k)
        def _():
            wsct(k)


@functools.partial(
    pl.kernel,
    out_type=jax.ShapeDtypeStruct((S, D), jnp.float32),
    mesh=plsc.VectorSubcoreMesh(core_axis_name="c", subcore_axis_name="s"),
    compiler_params=pltpu.CompilerParams(needs_layout_passes=False),
    scratch_types=[
        pltpu.VMEM((S,), jnp.int32),
        pltpu.VMEM((B, D), jnp.float32),
        pltpu.VMEM((B, D), jnp.float32),
        pltpu.VMEM((B, D), jnp.float32),
        pltpu.VMEM((B, D), jnp.float32),
        pltpu.VMEM((B, D), jnp.float32),
        pltpu.VMEM((NLR, B), jnp.int32),
        pltpu.VMEM((NLR, B), jnp.int32),
        pltpu.VMEM((NLR, B), jnp.int32),
        pltpu.VMEM((NLR, B), jnp.int32),
        pltpu.SemaphoreType.DMA,
        pltpu.SemaphoreType.DMA,
        pltpu.SemaphoreType.DMA,
        pltpu.SemaphoreType.DMA,
        pltpu.SemaphoreType.DMA,
        pltpu.SemaphoreType.DMA,
        pltpu.SemaphoreType.DMA,
        pltpu.SemaphoreType.DMA,
        pltpu.SemaphoreType.DMA,
        pltpu.SemaphoreType.DMA,
    ],
)
def _sc_masked_scatter(inputs_hbm, mask_hbm, source_hbm, out_hbm, *scratch):
    _body(inputs_hbm, mask_hbm, source_hbm, out_hbm, *scratch)


def kernel(inputs_embeds, mask_1d, source):
    mask_i32 = mask_1d.astype(jnp.int32)
    return _sc_masked_scatter(inputs_embeds, mask_i32, source)
